# Initial kernel scaffold; baseline (speedup 1.0000x reference)
import jax
import jax.numpy as jnp
from jax.experimental import pallas as pl


def kernel(x, edge_index, W_self, W_neigh, b_conv, W_proj, b_proj):
    raise NotImplementedError("write your pallas kernel here")


# speedup vs baseline: 3.1236x; 3.1236x over previous
import functools

import jax
import jax.numpy as jnp
from jax import lax
from jax.experimental import pallas as pl
from jax.experimental.pallas import tpu as pltpu
from jax.experimental.pallas import tpu_sc as plsc

N_NODES = 10000
N_EDGES = 320000
D_IN = 128
HIDDEN = 512

NC = 2
NS = 16
NW = NC * NS
EPW = N_EDGES // NW
CHUNK = 80
NCHUNK = EPW // CHUNK
DEG_W = 16

ZSTRIDE = 624
ZSIZE = 640


def _sc_agg(x, src, dst, zeros_agg):
    mesh = plsc.VectorSubcoreMesh(core_axis_name="c", subcore_axis_name="s")

    @functools.partial(
        pl.kernel,
        mesh=mesh,
        out_type=jax.ShapeDtypeStruct((NC, N_NODES, D_IN), jnp.float32),
        scratch_types=[
            pltpu.VMEM((CHUNK,), jnp.int32),
            pltpu.VMEM((CHUNK,), jnp.int32),
            pltpu.VMEM((CHUNK, D_IN), jnp.float32),
            pltpu.VMEM_SHARED((N_NODES, D_IN), jnp.float32),
            pltpu.SemaphoreType.DMA,
        ],
    )
    def k(x_hbm, src_hbm, dst_hbm, zagg_hbm, agg_out,
          src_v, dst_v, rows_v, agg_sh, sem):
        c = lax.axis_index("c")
        s = lax.axis_index("s")
        wid = c * NS + s
        row0 = s * ZSTRIDE

        pltpu.sync_copy(zagg_hbm.at[pl.ds(row0, ZSIZE)],
                        agg_sh.at[pl.ds(row0, ZSIZE)])
        plsc.subcore_barrier()

        base = wid * EPW

        def body(g, carry):
            off = base + g * CHUNK
            pltpu.sync_copy(src_hbm.at[pl.ds(off, CHUNK)], src_v)
            pltpu.sync_copy(dst_hbm.at[pl.ds(off, CHUNK)], dst_v)
            pltpu.async_copy(x_hbm.at[src_v], rows_v, sem).wait()
            pltpu.sync_copy(rows_v, agg_sh.at[dst_v], add=True)
            return carry
        lax.fori_loop(0, NCHUNK, body, 0)

        plsc.subcore_barrier()
        pltpu.sync_copy(agg_sh.at[pl.ds(row0, ZSIZE)],
                        agg_out.at[c, pl.ds(row0, ZSIZE)])

    return k(x, src, dst, zeros_agg)


def _unused_sc_deg(dst, zeros_deg):
    mesh = plsc.VectorSubcoreMesh(core_axis_name="c", subcore_axis_name="s")

    @functools.partial(
        pl.kernel,
        mesh=mesh,
        out_type=jax.ShapeDtypeStruct((NC, N_NODES, DEG_W), jnp.float32),
        scratch_types=[
            pltpu.VMEM((CHUNK,), jnp.int32),
            pltpu.VMEM((CHUNK, DEG_W), jnp.float32),
            pltpu.VMEM_SHARED((N_NODES, DEG_W), jnp.float32),
        ],
    )
    def k(dst_hbm, zdeg_hbm, deg_out, dst_v, ones_v, deg_sh):
        c = lax.axis_index("c")
        s = lax.axis_index("s")
        wid = c * NS + s
        row0 = s * ZSTRIDE

        pltpu.sync_copy(zdeg_hbm.at[pl.ds(row0, ZSIZE)],
                        deg_sh.at[pl.ds(row0, ZSIZE)])

        def fill_ones(i, carry):
            ones_v[i, :] = jnp.ones((DEG_W,), jnp.float32)
            return carry
        lax.fori_loop(0, CHUNK, fill_ones, 0)

        plsc.subcore_barrier()

        base = wid * EPW

        def body(g, carry):
            off = base + g * CHUNK
            pltpu.sync_copy(dst_hbm.at[pl.ds(off, CHUNK)], dst_v)
            pltpu.sync_copy(ones_v, deg_sh.at[dst_v], add=True)
            return carry
        lax.fori_loop(0, NCHUNK, body, 0)

        plsc.subcore_barrier()
        pltpu.sync_copy(deg_sh.at[pl.ds(row0, ZSIZE)],
                        deg_out.at[c, pl.ds(row0, ZSIZE)])

    return k(dst, zeros_deg)


def _tc_dense(x, agg2, deg2, W_self, W_neigh, b_conv, W_proj, b_proj):
    R = 400
    grid = (N_NODES // R,)

    def body(x_ref, agg_ref, deg_ref, ws_ref, wn_ref, bc_ref, wp_ref, bp_ref,
             out_ref):
        agg = agg_ref[0] + agg_ref[1]
        deg = (deg_ref[0] + deg_ref[1])[:, 0:1]
        mean = agg / jnp.maximum(deg, 1.0)
        h = jnp.dot(x_ref[...], ws_ref[...], preferred_element_type=jnp.float32)
        h = h + jnp.dot(mean, wn_ref[...], preferred_element_type=jnp.float32)
        h = jnp.maximum(h + bc_ref[...], 0.0)
        out_ref[...] = (jnp.dot(h, wp_ref[...],
                                preferred_element_type=jnp.float32)
                        + bp_ref[...])

    return pl.pallas_call(
        body,
        grid=grid,
        in_specs=[
            pl.BlockSpec((R, D_IN), lambda i: (i, 0)),
            pl.BlockSpec((NC, R, D_IN), lambda i: (0, i, 0)),
            pl.BlockSpec((NC, R, DEG_W), lambda i: (0, i, 0)),
            pl.BlockSpec((D_IN, HIDDEN), lambda i: (0, 0)),
            pl.BlockSpec((D_IN, HIDDEN), lambda i: (0, 0)),
            pl.BlockSpec((1, HIDDEN), lambda i: (0, 0)),
            pl.BlockSpec((HIDDEN, HIDDEN), lambda i: (0, 0)),
            pl.BlockSpec((1, HIDDEN), lambda i: (0, 0)),
        ],
        out_specs=pl.BlockSpec((R, HIDDEN), lambda i: (i, 0)),
        out_shape=jax.ShapeDtypeStruct((N_NODES, HIDDEN), jnp.float32),
    )(x, agg2, deg2, W_self, W_neigh, b_conv, W_proj, b_proj)


def kernel(x, edge_index, W_self, W_neigh, b_conv, W_proj, b_proj):
    src = edge_index[0]
    dst = edge_index[1]
    zeros_agg = jnp.zeros((N_NODES, D_IN), jnp.float32)
    agg2 = _sc_agg(x, src, dst, zeros_agg)
    deg1 = jax.ops.segment_sum(jnp.ones((N_EDGES,), jnp.float32), dst,
                               num_segments=N_NODES)
    deg2 = jnp.broadcast_to(deg1[None, :, None], (NC, N_NODES, DEG_W)) * 0.5
    return _tc_dense(x, agg2, deg2, W_self, W_neigh,
                     b_conv.reshape(1, HIDDEN), W_proj,
                     b_proj.reshape(1, HIDDEN))


# speedup vs baseline: 4.6125x; 1.4767x over previous
import functools

import jax
import jax.numpy as jnp
from jax import lax
from jax.experimental import pallas as pl
from jax.experimental.pallas import tpu as pltpu
from jax.experimental.pallas import tpu_sc as plsc

N_NODES = 10000
N_EDGES = 320000
D_IN = 128
HIDDEN = 512

NC = 2
NS = 16
NW = NC * NS
EPW = N_EDGES // NW
CHUNK = 80
NCHUNK = EPW // CHUNK
DEG_W = 16

ZSTRIDE = 624
ZSIZE = 640


def _sc_agg(x, src, dst, zeros_agg):
    mesh = plsc.VectorSubcoreMesh(core_axis_name="c", subcore_axis_name="s")

    @functools.partial(
        pl.kernel,
        mesh=mesh,
        out_type=jax.ShapeDtypeStruct((NC, N_NODES, D_IN), jnp.float32),
        scratch_types=[
            pltpu.VMEM((CHUNK,), jnp.int32),
            pltpu.VMEM((CHUNK,), jnp.int32),
            pltpu.VMEM((CHUNK, D_IN), jnp.float32),
            pltpu.VMEM_SHARED((N_NODES, D_IN), jnp.float32),
            pltpu.SemaphoreType.DMA,
        ],
    )
    def k(x_hbm, src_hbm, dst_hbm, zagg_hbm, agg_out,
          src_v, dst_v, rows_v, agg_sh, sem):
        c = lax.axis_index("c")
        s = lax.axis_index("s")
        wid = c * NS + s
        row0 = s * ZSTRIDE

        pltpu.sync_copy(zagg_hbm.at[pl.ds(row0, ZSIZE)],
                        agg_sh.at[pl.ds(row0, ZSIZE)])
        plsc.subcore_barrier()

        base = wid * EPW

        def body(g, carry):
            off = base + g * CHUNK
            pltpu.sync_copy(src_hbm.at[pl.ds(off, CHUNK)], src_v)
            pltpu.sync_copy(dst_hbm.at[pl.ds(off, CHUNK)], dst_v)
            pltpu.async_copy(x_hbm.at[src_v], rows_v, sem).wait()
            pltpu.sync_copy(rows_v, agg_sh.at[dst_v], add=True)
            return carry
        lax.fori_loop(0, NCHUNK, body, 0)

        plsc.subcore_barrier()
        pltpu.sync_copy(agg_sh.at[pl.ds(row0, ZSIZE)],
                        agg_out.at[c, pl.ds(row0, ZSIZE)])

    return k(x, src, dst, zeros_agg)


def _sc_deg(dst, zeros_agg):
    mesh = plsc.VectorSubcoreMesh(core_axis_name="c", subcore_axis_name="s")

    @functools.partial(
        pl.kernel,
        mesh=mesh,
        out_type=jax.ShapeDtypeStruct((NC, N_NODES, D_IN), jnp.float32),
        scratch_types=[
            pltpu.VMEM((CHUNK,), jnp.int32),
            pltpu.VMEM((CHUNK, D_IN), jnp.float32),
            pltpu.VMEM_SHARED((N_NODES, D_IN), jnp.float32),
        ],
    )
    def k(dst_hbm, zagg_hbm, deg_out, dst_v, ones_v, deg_sh):
        c = lax.axis_index("c")
        s = lax.axis_index("s")
        wid = c * NS + s
        row0 = s * ZSTRIDE

        pltpu.sync_copy(zagg_hbm.at[pl.ds(row0, ZSIZE)],
                        deg_sh.at[pl.ds(row0, ZSIZE)])

        one16 = jnp.ones((16,), jnp.float32)

        def fill_ones(i, carry):
            def fill_col(j, carry2):
                ones_v[i, pl.ds(j * 16, 16)] = one16
                return carry2
            return lax.fori_loop(0, D_IN // 16, fill_col, carry)
        lax.fori_loop(0, CHUNK, fill_ones, 0)

        plsc.subcore_barrier()

        base = wid * EPW

        def body(g, carry):
            off = base + g * CHUNK
            pltpu.sync_copy(dst_hbm.at[pl.ds(off, CHUNK)], dst_v)
            pltpu.sync_copy(ones_v, deg_sh.at[dst_v], add=True)
            return carry
        lax.fori_loop(0, NCHUNK, body, 0)

        plsc.subcore_barrier()
        pltpu.sync_copy(deg_sh.at[pl.ds(row0, ZSIZE)],
                        deg_out.at[c, pl.ds(row0, ZSIZE)])

    return k(dst, zeros_agg)


def _tc_dense(x, agg2, deg2, W_self, W_neigh, b_conv, W_proj, b_proj):
    R = 400
    grid = (N_NODES // R,)

    def body(x_ref, agg_ref, deg_ref, ws_ref, wn_ref, bc_ref, wp_ref, bp_ref,
             out_ref):
        agg = agg_ref[0] + agg_ref[1]
        deg = (deg_ref[0] + deg_ref[1])[:, 0:1]
        mean = agg / jnp.maximum(deg, 1.0)
        h = jnp.dot(x_ref[...], ws_ref[...], preferred_element_type=jnp.float32)
        h = h + jnp.dot(mean, wn_ref[...], preferred_element_type=jnp.float32)
        h = jnp.maximum(h + bc_ref[...], 0.0)
        out_ref[...] = (jnp.dot(h, wp_ref[...],
                                preferred_element_type=jnp.float32)
                        + bp_ref[...])

    return pl.pallas_call(
        body,
        grid=grid,
        in_specs=[
            pl.BlockSpec((R, D_IN), lambda i: (i, 0)),
            pl.BlockSpec((NC, R, D_IN), lambda i: (0, i, 0)),
            pl.BlockSpec((NC, R, D_IN), lambda i: (0, i, 0)),
            pl.BlockSpec((D_IN, HIDDEN), lambda i: (0, 0)),
            pl.BlockSpec((D_IN, HIDDEN), lambda i: (0, 0)),
            pl.BlockSpec((1, HIDDEN), lambda i: (0, 0)),
            pl.BlockSpec((HIDDEN, HIDDEN), lambda i: (0, 0)),
            pl.BlockSpec((1, HIDDEN), lambda i: (0, 0)),
        ],
        out_specs=pl.BlockSpec((R, HIDDEN), lambda i: (i, 0)),
        out_shape=jax.ShapeDtypeStruct((N_NODES, HIDDEN), jnp.float32),
    )(x, agg2, deg2, W_self, W_neigh, b_conv, W_proj, b_proj)


def kernel(x, edge_index, W_self, W_neigh, b_conv, W_proj, b_proj):
    src = edge_index[0]
    dst = edge_index[1]
    zeros_agg = jnp.zeros((N_NODES, D_IN), jnp.float32)
    agg2 = _sc_agg(x, src, dst, zeros_agg)
    deg2 = _sc_deg(dst, zeros_agg)
    return _tc_dense(x, agg2, deg2, W_self, W_neigh,
                     b_conv.reshape(1, HIDDEN), W_proj,
                     b_proj.reshape(1, HIDDEN))
